# baseline (speedup 1.0000x reference)
import functools

import jax
import jax.numpy as jnp
from jax import lax
from jax.experimental import pallas as pl
from jax.experimental.pallas import tpu as pltpu
from jax.experimental.pallas import tpu_sc as plsc

NC = 2
NS = 16
NW = NC * NS
CH = 128

N = 10000
S = 24
F = 32
N_T = N + 16
N_A = 10240
ZROWS = N_A // NS

_HI = jax.lax.Precision.HIGHEST


def _make_sc_agg(K):
    mesh = plsc.VectorSubcoreMesh(core_axis_name="c", subcore_axis_name="s")

    @functools.partial(
        pl.kernel,
        out_type=jax.ShapeDtypeStruct((NC * N_A, F), jnp.float32),
        mesh=mesh,
        compiler_params=pltpu.CompilerParams(use_tc_tiling_on_sc=False),
        scratch_types=[
            pltpu.VMEM((K, CH), jnp.int32),
            pltpu.VMEM((K, CH), jnp.int32),
            pltpu.VMEM((CH, F), jnp.float32),
            pltpu.VMEM((CH, F), jnp.float32),
            pltpu.VMEM((CH, F), jnp.float32),
            pltpu.VMEM((CH, F), jnp.float32),
            pltpu.VMEM((ZROWS, F), jnp.float32),
            pltpu.VMEM_SHARED((N_A, F), jnp.float32),
            pltpu.SemaphoreType.DMA,
            pltpu.SemaphoreType.DMA,
            pltpu.SemaphoreType.DMA,
            pltpu.SemaphoreType.DMA,
            pltpu.SemaphoreType.DMA,
            pltpu.SemaphoreType.DMA,
            pltpu.SemaphoreType.DMA,
            pltpu.SemaphoreType.DMA,
        ],
    )
    def sc_agg(edges_hbm, table_hbm, out_hbm,
               src_v, dst_v, rows_a, rows_b, rows_c, rows_d, stage_v, acc_sh,
               ga, gb, gc, gd, sa, sb, sc, sd):
        cid = lax.axis_index("c")
        sid = lax.axis_index("s")
        wid = sid * NC + cid

        z16 = jnp.zeros((16,), jnp.float32)

        def _zrow(i, carry):
            stage_v[i, pl.ds(0, 16)] = z16
            stage_v[i, pl.ds(F - 16, 16)] = z16
            return carry

        lax.fori_loop(0, ZROWS, _zrow, 0)
        pltpu.sync_copy(stage_v, acc_sh.at[pl.ds(sid * ZROWS, ZROWS)])

        pltpu.sync_copy(edges_hbm.at[0, pl.ds(wid * K, K)], src_v)
        pltpu.sync_copy(edges_hbm.at[1, pl.ds(wid * K, K)], dst_v)

        plsc.subcore_barrier()

        bufs = (rows_a, rows_b, rows_c, rows_d)
        gsems = (ga, gb, gc, gd)
        ssems = (sa, sb, sc, sd)

        def _gather(j, i):
            pltpu.async_copy(table_hbm.at[src_v.at[j]], bufs[i], gsems[i])

        def _wait_gather(j, i):
            pltpu.make_async_copy(
                table_hbm.at[src_v.at[j]], bufs[i], gsems[i]).wait()

        def _scatter(j, i):
            pltpu.async_copy(bufs[i], acc_sh.at[dst_v.at[j]], ssems[i],
                             add=True)

        def _wait_scatter(j, i):
            pltpu.make_async_copy(
                bufs[i], acc_sh.at[dst_v.at[j]], ssems[i]).wait()

        for i in range(4):
            _gather(i, i)

        def _quad(qq, carry):
            j = 4 * qq
            for i in range(4):
                _wait_gather(j + i, i)
                _scatter(j + i, i)
                _wait_scatter(j + i, i)
                _gather(j + 4 + i, i)
            return carry

        lax.fori_loop(0, K // 4 - 1, _quad, 0)
        for i in range(4):
            _wait_gather(K - 4 + i, i)
            _scatter(K - 4 + i, i)
            _wait_scatter(K - 4 + i, i)

        plsc.subcore_barrier()

        pltpu.sync_copy(acc_sh.at[pl.ds(sid * ZROWS, ZROWS)], stage_v)
        pltpu.sync_copy(stage_v, out_hbm.at[pl.ds(cid * N_A + sid * ZROWS, ZROWS)])

    return sc_agg


def _tbl_body(x_ref, tbl_ref):
    x0 = x_ref[0]
    eye = jnp.eye(S, dtype=jnp.float32)
    xt = lax.dot_general(x0, eye, (((0,), (0,)), ((), ())), precision=_HI)
    tbl = jnp.concatenate(
        [xt, jnp.ones((N, 1), jnp.float32),
         jnp.zeros((N, F - S - 1), jnp.float32)], axis=1)
    tbl = jnp.concatenate(
        [tbl, jnp.zeros((N_T - N, F), jnp.float32)], axis=0)
    tbl_ref[...] = tbl


def _fin_rest_body(x_ref, wr_ref, bl_ref, out_ref):
    out_ref[0] = x_ref[0] * wr_ref[0, 0] + bl_ref[0]


def _fin_b0_body(rest_ref, parts_ref, wl_ref, out_ref):
    p = parts_ref[...]
    comb = p[0:N, :] + p[N_A:N_A + N, :]
    mean_nf = comb[:, 0:S] / jnp.maximum(comb[:, S:S + 1], 1.0)
    eye = jnp.eye(S, dtype=jnp.float32)
    mean_t = lax.dot_general(
        eye, mean_nf, (((1,), (1,)), ((), ())), precision=_HI)
    out_ref[0] = rest_ref[0] + wl_ref[0, 0] * mean_t


def kernel(x, edge_index, W_l, W_r, b_l):
    B, S_, N_ = x.shape
    E = edge_index.shape[1]
    K = -(-E // (NW * CH))
    E_pad = NW * K * CH

    table = pl.pallas_call(
        _tbl_body,
        grid=(1,),
        in_specs=[pl.BlockSpec((1, S_, N_), lambda i: (0, 0, 0))],
        out_specs=pl.BlockSpec((N_T, F), lambda i: (0, 0)),
        out_shape=jax.ShapeDtypeStruct((N_T, F), jnp.float32),
    )(x)

    npad = E_pad - E
    pad_src = (N_ + (jnp.arange(npad, dtype=jnp.int32) % (N_T - N_)))[None]
    pad_dst = (jnp.arange(npad, dtype=jnp.int32) % N_)[None]
    edges = jnp.concatenate(
        [edge_index, jnp.concatenate([pad_src, pad_dst], axis=0)], axis=1)
    edges = edges.reshape(2, NW * K, CH)
    parts = _make_sc_agg(K)(edges, table)

    rest = pl.pallas_call(
        _fin_rest_body,
        grid=(B,),
        in_specs=[
            pl.BlockSpec((1, S_, N_), lambda b: (b, 0, 0)),
            pl.BlockSpec(memory_space=pltpu.SMEM),
            pl.BlockSpec(memory_space=pltpu.SMEM),
        ],
        out_specs=pl.BlockSpec((1, S_, N_), lambda b: (b, 0, 0)),
        out_shape=jax.ShapeDtypeStruct((B, S_, N_), jnp.float32),
    )(x, W_r, b_l)

    out = pl.pallas_call(
        _fin_b0_body,
        grid=(1,),
        in_specs=[
            pl.BlockSpec((1, S_, N_), lambda i: (0, 0, 0)),
            pl.BlockSpec((NC * N_A, F), lambda i: (0, 0)),
            pl.BlockSpec(memory_space=pltpu.SMEM),
        ],
        out_specs=pl.BlockSpec((1, S_, N_), lambda i: (0, 0, 0)),
        out_shape=jax.ShapeDtypeStruct((B, S_, N_), jnp.float32),
        input_output_aliases={0: 0},
    )(rest, parts, W_l)
    return out


# speedup vs baseline: 1.0564x; 1.0564x over previous
import functools

import jax
import jax.numpy as jnp
from jax import lax
from jax.experimental import pallas as pl
from jax.experimental.pallas import tpu as pltpu
from jax.experimental.pallas import tpu_sc as plsc

NC = 2
NS = 16
NW = NC * NS
CH = 125

N = 10000
S = 24
F = 32
N_T = N + 16
N_A = 10240
ZROWS = N_A // NS

_HI = jax.lax.Precision.HIGHEST


def _make_sc_agg(K):
    mesh = plsc.VectorSubcoreMesh(core_axis_name="c", subcore_axis_name="s")

    @functools.partial(
        pl.kernel,
        out_type=jax.ShapeDtypeStruct((NC * N_A, F), jnp.float32),
        mesh=mesh,
        compiler_params=pltpu.CompilerParams(use_tc_tiling_on_sc=False),
        scratch_types=[
            pltpu.VMEM((K, CH), jnp.int32),
            pltpu.VMEM((K, CH), jnp.int32),
            pltpu.VMEM((CH, F), jnp.float32),
            pltpu.VMEM((CH, F), jnp.float32),
            pltpu.VMEM((CH, F), jnp.float32),
            pltpu.VMEM((CH, F), jnp.float32),
            pltpu.VMEM((ZROWS, F), jnp.float32),
            pltpu.VMEM_SHARED((N_A, F), jnp.float32),
            pltpu.SemaphoreType.DMA,
            pltpu.SemaphoreType.DMA,
            pltpu.SemaphoreType.DMA,
            pltpu.SemaphoreType.DMA,
            pltpu.SemaphoreType.DMA,
            pltpu.SemaphoreType.DMA,
            pltpu.SemaphoreType.DMA,
            pltpu.SemaphoreType.DMA,
        ],
    )
    def sc_agg(edges_hbm, table_hbm, out_hbm,
               src_v, dst_v, rows_a, rows_b, rows_c, rows_d, stage_v, acc_sh,
               ga, gb, gc, gd, sa, sb, sc, sd):
        cid = lax.axis_index("c")
        sid = lax.axis_index("s")
        wid = sid * NC + cid

        z16 = jnp.zeros((16,), jnp.float32)

        def _zrow(i, carry):
            stage_v[i, pl.ds(0, 16)] = z16
            stage_v[i, pl.ds(F - 16, 16)] = z16
            return carry

        lax.fori_loop(0, ZROWS, _zrow, 0)
        pltpu.sync_copy(stage_v, acc_sh.at[pl.ds(sid * ZROWS, ZROWS)])

        pltpu.sync_copy(edges_hbm.at[0, pl.ds(wid * K, K)], src_v)
        pltpu.sync_copy(edges_hbm.at[1, pl.ds(wid * K, K)], dst_v)

        plsc.subcore_barrier()

        bufs = (rows_a, rows_b, rows_c, rows_d)
        gsems = (ga, gb, gc, gd)
        ssems = (sa, sb, sc, sd)

        def _gather(j, i):
            pltpu.async_copy(table_hbm.at[src_v.at[j]], bufs[i], gsems[i])

        def _wait_gather(j, i):
            pltpu.make_async_copy(
                table_hbm.at[src_v.at[j]], bufs[i], gsems[i]).wait()

        def _scatter(j, i):
            pltpu.async_copy(bufs[i], acc_sh.at[dst_v.at[j]], ssems[i],
                             add=True)

        def _wait_scatter(j, i):
            pltpu.make_async_copy(
                bufs[i], acc_sh.at[dst_v.at[j]], ssems[i]).wait()

        for i in range(4):
            _gather(i, i)

        def _quad(qq, carry):
            j = 4 * qq
            for i in range(4):
                _wait_gather(j + i, i)
                _scatter(j + i, i)
                _wait_scatter(j + i, i)
                _gather(j + 4 + i, i)
            return carry

        lax.fori_loop(0, K // 4 - 1, _quad, 0)
        for i in range(4):
            _wait_gather(K - 4 + i, i)
            _scatter(K - 4 + i, i)
            _wait_scatter(K - 4 + i, i)

        plsc.subcore_barrier()

        pltpu.sync_copy(acc_sh.at[pl.ds(sid * ZROWS, ZROWS)], stage_v)
        pltpu.sync_copy(stage_v, out_hbm.at[pl.ds(cid * N_A + sid * ZROWS, ZROWS)])

    return sc_agg


def _tbl_body(x_ref, tbl_ref):
    x0 = x_ref[0]
    eye = jnp.eye(S, dtype=jnp.float32)
    xt = lax.dot_general(x0, eye, (((0,), (0,)), ((), ())), precision=_HI)
    tbl = jnp.concatenate(
        [xt, jnp.ones((N, 1), jnp.float32),
         jnp.zeros((N, F - S - 1), jnp.float32)], axis=1)
    tbl = jnp.concatenate(
        [tbl, jnp.zeros((N_T - N, F), jnp.float32)], axis=0)
    tbl_ref[...] = tbl


def _fin_rest_body(x_ref, wr_ref, bl_ref, out_ref):
    out_ref[0] = x_ref[0] * wr_ref[0, 0] + bl_ref[0]


def _fin_b0_body(rest_ref, parts_ref, wl_ref, out_ref):
    p = parts_ref[...]
    comb = p[0:N, :] + p[N_A:N_A + N, :]
    mean_nf = comb[:, 0:S] / jnp.maximum(comb[:, S:S + 1], 1.0)
    eye = jnp.eye(S, dtype=jnp.float32)
    mean_t = lax.dot_general(
        eye, mean_nf, (((1,), (1,)), ((), ())), precision=_HI)
    out_ref[0] = rest_ref[0] + wl_ref[0, 0] * mean_t


def kernel(x, edge_index, W_l, W_r, b_l):
    B, S_, N_ = x.shape
    E = edge_index.shape[1]
    assert E % (NW * CH) == 0
    K = E // (NW * CH)

    table = pl.pallas_call(
        _tbl_body,
        grid=(1,),
        in_specs=[pl.BlockSpec((1, S_, N_), lambda i: (0, 0, 0))],
        out_specs=pl.BlockSpec((N_T, F), lambda i: (0, 0)),
        out_shape=jax.ShapeDtypeStruct((N_T, F), jnp.float32),
    )(x)

    edges = edge_index.reshape(2, NW * K, CH)
    parts = _make_sc_agg(K)(edges, table)

    rest = pl.pallas_call(
        _fin_rest_body,
        grid=(B,),
        in_specs=[
            pl.BlockSpec((1, S_, N_), lambda b: (b, 0, 0)),
            pl.BlockSpec(memory_space=pltpu.SMEM),
            pl.BlockSpec(memory_space=pltpu.SMEM),
        ],
        out_specs=pl.BlockSpec((1, S_, N_), lambda b: (b, 0, 0)),
        out_shape=jax.ShapeDtypeStruct((B, S_, N_), jnp.float32),
    )(x, W_r, b_l)

    out = pl.pallas_call(
        _fin_b0_body,
        grid=(1,),
        in_specs=[
            pl.BlockSpec((1, S_, N_), lambda i: (0, 0, 0)),
            pl.BlockSpec((NC * N_A, F), lambda i: (0, 0)),
            pl.BlockSpec(memory_space=pltpu.SMEM),
        ],
        out_specs=pl.BlockSpec((1, S_, N_), lambda i: (0, 0, 0)),
        out_shape=jax.ShapeDtypeStruct((B, S_, N_), jnp.float32),
        input_output_aliases={0: 0},
    )(rest, parts, W_l)
    return out
